# baseline (speedup 1.0000x reference)
import functools

import jax
import jax.numpy as jnp
from jax import lax
from jax.experimental import pallas as pl
from jax.experimental.pallas import tpu as pltpu
from jax.experimental.pallas import tpu_sc as plsc

NC, NS, L = 2, 16, 16
NW = NC * NS
CB = 128
TRASH = 128
HID = 128
NHEAD, CDIM = 4, 32
BN = 512

_mesh = functools.partial(
    plsc.VectorSubcoreMesh, core_axis_name="c", subcore_axis_name="s")


def _ceil_to(x, m):
    return ((x + m - 1) // m) * m


@functools.lru_cache(maxsize=None)
def _gather(N, D, E):
    cnt = E // NW
    n_chunks = cnt // CB

    def body(table, idx, out, idx_v, rows_v, sem):
        wid = lax.axis_index("s") * NC + lax.axis_index("c")
        base = wid * cnt

        def chunk(i, carry):
            off = pl.multiple_of(base + i * CB, CB)
            pltpu.sync_copy(idx.at[pl.ds(off, CB)], idx_v)
            pltpu.async_copy(table.at[idx_v], rows_v, sem).wait()
            pltpu.sync_copy(rows_v, out.at[pl.ds(off, CB)])
            return carry

        lax.fori_loop(0, n_chunks, chunk, 0)

    return pl.kernel(
        body,
        out_type=jax.ShapeDtypeStruct((E, D), jnp.float32),
        mesh=_mesh(),
        scratch_types=[
            pltpu.VMEM((CB,), jnp.int32),
            pltpu.VMEM((CB, D), jnp.float32),
            pltpu.SemaphoreType.DMA,
        ],
    )


@functools.lru_cache(maxsize=None)
def _scatter(N, D, E, R):
    n_ranges = N // R
    assert n_ranges % NC == 0, (N, R)
    n_ri = n_ranges // NC
    cnt = E // NS
    n_chunks = cnt // CB
    out_rpt = R // NS
    zslice = (R + TRASH) // NS

    def body(rows, dst, zeros, out, idx_v, lidx_v, rows_v, acc):
        c = lax.axis_index("c")
        s = lax.axis_index("s")
        for ri in range(n_ri):
            r = ri * NC + c
            base = r * R

            zo = pl.multiple_of(s * zslice, 8)
            pltpu.sync_copy(zeros.at[pl.ds(zo, zslice)],
                            acc.at[pl.ds(zo, zslice)])

            plsc.subcore_barrier()

            def chunk(i, carry):
                off = pl.multiple_of(s * cnt + i * CB, CB)
                pltpu.sync_copy(dst.at[pl.ds(off, CB)], idx_v)
                pltpu.sync_copy(rows.at[pl.ds(off, CB)], rows_v)

                def fix(j, carry2):
                    dv = idx_v[pl.ds(j * L, L)]
                    lv = dv - base
                    inb = (lv >= 0) & (lv < R)
                    lidx_v[pl.ds(j * L, L)] = jnp.where(inb, lv, R + s)
                    return carry2

                lax.fori_loop(0, CB // L, fix, 0)
                pltpu.sync_copy(rows_v, acc.at[lidx_v], add=True)
                return carry

            lax.fori_loop(0, n_chunks, chunk, 0)

            plsc.subcore_barrier()

            o = pl.multiple_of(s * out_rpt, 8)
            oo = pl.multiple_of(base + s * out_rpt, 8)
            pltpu.sync_copy(acc.at[pl.ds(o, out_rpt)],
                            out.at[pl.ds(oo, out_rpt)])

            plsc.subcore_barrier()

    return pl.kernel(
        body,
        out_type=jax.ShapeDtypeStruct((N, D), jnp.float32),
        mesh=_mesh(),
        scratch_types=[
            pltpu.VMEM((CB,), jnp.int32),
            pltpu.VMEM((CB,), jnp.int32),
            pltpu.VMEM((CB, D), jnp.float32),
            pltpu.VMEM_SHARED((R + TRASH, D), jnp.float32),
        ],
    )


@functools.lru_cache(maxsize=None)
def _scatter_c16(N, E, R):
    n_ranges = N // R
    assert n_ranges % NC == 0, (N, R)
    n_ri = n_ranges // NC
    cnt = E // NS
    CCB = 64
    n_chunks = cnt // CCB
    out_rpt = R // NS
    zslice = (R + TRASH) // NS

    def body(ex, dst, zeros, out, idx_v, lidx_v, ex_v, rows_v, acc):
        c = lax.axis_index("c")
        s = lax.axis_index("s")
        pltpu.sync_copy(zeros.at[pl.ds(0, CCB)], rows_v)
        for ri in range(n_ri):
            r = ri * NC + c
            base = r * R

            zo = pl.multiple_of(s * zslice, 8)
            pltpu.sync_copy(zeros.at[pl.ds(zo, zslice)],
                            acc.at[pl.ds(zo, zslice)])

            plsc.subcore_barrier()

            def chunk(i, carry):
                off = pl.multiple_of(s * cnt + i * CCB, CCB)
                pltpu.sync_copy(dst.at[pl.ds(off, CCB)], idx_v)
                pltpu.sync_copy(ex.at[pl.ds(off, CCB)], ex_v)

                def fix(j, carry2):
                    dv = idx_v[pl.ds(j * L, L)]
                    lv = dv - base
                    inb = (lv >= 0) & (lv < R)
                    lidx_v[pl.ds(j * L, L)] = jnp.where(inb, lv, R + s)
                    return carry2

                lax.fori_loop(0, CCB // L, fix, 0)
                for k in range(CCB):
                    rows_v[k, pl.ds(0, L)] = ex_v[k, pl.ds(0, L)]
                pltpu.sync_copy(rows_v, acc.at[lidx_v], add=True)
                return carry

            lax.fori_loop(0, n_chunks, chunk, 0)

            plsc.subcore_barrier()

            o = pl.multiple_of(s * out_rpt, 8)
            oo = pl.multiple_of(base + s * out_rpt, 8)
            pltpu.sync_copy(acc.at[pl.ds(o, out_rpt)],
                            out.at[pl.ds(oo, out_rpt)])

            plsc.subcore_barrier()

    return pl.kernel(
        body,
        out_type=jax.ShapeDtypeStruct((N, HID), jnp.float32),
        mesh=_mesh(),
        scratch_types=[
            pltpu.VMEM((CCB,), jnp.int32),
            pltpu.VMEM((CCB,), jnp.int32),
            pltpu.VMEM((CCB, L), jnp.float32),
            pltpu.VMEM((CCB, HID), jnp.float32),
            pltpu.VMEM_SHARED((R + TRASH, HID), jnp.float32),
        ],
    )


def _mm_block(x_ref, w_ref, b_ref, o_ref):
    o_ref[...] = jnp.dot(x_ref[...], w_ref[...],
                         preferred_element_type=jnp.float32) + b_ref[...]


@functools.lru_cache(maxsize=None)
def _mm(N, K):
    return pl.pallas_call(
        _mm_block,
        grid=(N // BN,),
        in_specs=[pl.BlockSpec((BN, K), lambda i: (i, 0)),
                  pl.BlockSpec((K, HID), lambda i: (0, 0)),
                  pl.BlockSpec((1, HID), lambda i: (0, 0))],
        out_specs=pl.BlockSpec((BN, HID), lambda i: (i, 0)),
        out_shape=jax.ShapeDtypeStruct((N, HID), jnp.float32),
    )


def _mms_block(x_ref, w_ref, s_ref, o_ref):
    o_ref[...] = jnp.dot(x_ref[...], w_ref[...],
                         preferred_element_type=jnp.float32) * s_ref[:, 0:1]


@functools.lru_cache(maxsize=None)
def _mm_scale(N, K):
    return pl.pallas_call(
        _mms_block,
        grid=(N // BN,),
        in_specs=[pl.BlockSpec((BN, K), lambda i: (i, 0)),
                  pl.BlockSpec((K, HID), lambda i: (0, 0)),
                  pl.BlockSpec((BN, L), lambda i: (i, 0))],
        out_specs=pl.BlockSpec((BN, HID), lambda i: (i, 0)),
        out_shape=jax.ShapeDtypeStruct((N, HID), jnp.float32),
    )


def _ew(d_in, d_par, d_out, fn, N):
    def block(*refs):
        o_ref = refs[-1]
        o_ref[...] = fn(*[r[...] for r in refs[:-1]])

    return pl.pallas_call(
        block,
        grid=(N // BN,),
        in_specs=([pl.BlockSpec((BN, d), lambda i: (i, 0)) for d in d_in]
                  + [pl.BlockSpec((1, d), lambda i: (0, 0)) for d in d_par]),
        out_specs=pl.BlockSpec((BN, d_out), lambda i: (i, 0)),
        out_shape=jax.ShapeDtypeStruct((N, d_out), jnp.float32),
    )


def _heads_reduce(rows, a):
    p = rows * a
    return [jnp.sum(p[:, i * CDIM:(i + 1) * CDIM], axis=1, keepdims=True)
            for i in range(NHEAD)]


def _heads_expand(v):
    return jnp.concatenate(
        [jnp.broadcast_to(v[:, i:i + 1], (v.shape[0], CDIM))
         for i in range(NHEAD)], axis=1)


def _f_ex16(hsr, hdr, a_s, a_d):
    es = _heads_reduce(hsr, a_s)
    ed = _heads_reduce(hdr, a_d)
    cols = [s + d for s, d in zip(es, ed)]
    cols.append(jnp.zeros((hsr.shape[0], L - NHEAD), jnp.float32))
    a = jnp.concatenate(cols, axis=1)
    return jnp.exp(jnp.where(a > 0, a, 0.2 * a))


def _f_num(hsr, ex):
    return hsr * _heads_expand(ex)


def _f_dinv(deg):
    return lax.rsqrt(deg[:, :L] + 1.0)


def _gat_norm(num, den):
    return num * _heads_expand(1.0 / (den + 1e-16))


def _f_cpat(s_gcn, xwn, di, num_rh, den_rh, num_rd, den_rd, b1, b2, b3):
    d = di[:, 0:1]
    return jnp.maximum(
        (s_gcn + xwn) * d + _gat_norm(num_rh, den_rh)
        + _gat_norm(num_rd, den_rd) + b1 + b2 + b3, 0.0)


def _f_crelu(num, den, b):
    return jnp.maximum(_gat_norm(num, den) + b, 0.0)


_NPAD = {"patient": 51200, "signature": 20480, "condition": 1024}
_NREAL = {"patient": 50000, "signature": 20000, "condition": 500}
_R128 = {"patient": 12800, "signature": 10240, "condition": 512}
_GAT_RELS = [("has", "patient", "signature"),
             ("rev_has", "signature", "patient"),
             ("diagnosed", "patient", "condition"),
             ("rev_diagnosed", "condition", "patient")]


def kernel(x_patient, x_signature, x_condition, edge_follows, edge_has, edge_rev_has, edge_diagnosed, edge_rev_diagnosed, proj_patient_W, proj_patient_b, out_patient_W, out_patient_b, proj_signature_W, proj_signature_b, out_signature_W, out_signature_b, proj_condition_W, proj_condition_b, out_condition_W, out_condition_b, l1_follows_W, l1_follows_b, l1_has_Ws, l1_has_Wd, l1_has_as, l1_has_ad, l1_has_b, l1_rev_has_Ws, l1_rev_has_Wd, l1_rev_has_as, l1_rev_has_ad, l1_rev_has_b, l1_diagnosed_Ws, l1_diagnosed_Wd, l1_diagnosed_as, l1_diagnosed_ad, l1_diagnosed_b, l1_rev_diagnosed_Ws, l1_rev_diagnosed_Wd, l1_rev_diagnosed_as, l1_rev_diagnosed_ad, l1_rev_diagnosed_b, l2_follows_W, l2_follows_b, l2_has_Ws, l2_has_Wd, l2_has_as, l2_has_ad, l2_has_b, l2_rev_has_Ws, l2_rev_has_Wd, l2_rev_has_as, l2_rev_has_ad, l2_rev_has_b, l2_diagnosed_Ws, l2_diagnosed_Wd, l2_diagnosed_as, l2_diagnosed_ad, l2_diagnosed_b, l2_rev_diagnosed_Ws, l2_rev_diagnosed_Wd, l2_rev_diagnosed_as, l2_rev_diagnosed_ad, l2_rev_diagnosed_b, l3_follows_W, l3_follows_b, l3_has_Ws, l3_has_Wd, l3_has_as, l3_has_ad, l3_has_b, l3_rev_has_Ws, l3_rev_has_Wd, l3_rev_has_as, l3_rev_has_ad, l3_rev_has_b, l3_diagnosed_Ws, l3_diagnosed_Wd, l3_diagnosed_as, l3_diagnosed_ad, l3_diagnosed_b, l3_rev_diagnosed_Ws, l3_rev_diagnosed_Wd, l3_rev_diagnosed_as, l3_rev_diagnosed_ad, l3_rev_diagnosed_b):
    kw = dict(locals())
    node_types = ["patient", "signature", "condition"]

    edges = {}
    for rel in ("follows", "has", "rev_has", "diagnosed", "rev_diagnosed"):
        ei = kw[f"edge_{rel}"].astype(jnp.int32)
        e = ei.shape[1]
        ep = _ceil_to(e, NW * CB)
        src = jnp.pad(ei[0], (0, ep - e))
        dstg = jnp.pad(ei[1], (0, ep - e))
        dsts = jnp.pad(ei[1], (0, ep - e), constant_values=-1)
        edges[rel] = (src, dstg, dsts, ep)

    zb = jnp.zeros((1, HID), jnp.float32)
    z128 = {nt: jnp.zeros((_R128[nt] + TRASH, HID), jnp.float32)
            for nt in node_types}

    x = {}
    for nt in node_types:
        xin = kw[f"x_{nt}"]
        k = _ceil_to(xin.shape[1], HID)
        xp = jnp.pad(xin, ((0, _NPAD[nt] - xin.shape[0]),
                           (0, k - xin.shape[1])))
        wp = jnp.pad(kw[f"proj_{nt}_W"], ((0, k - xin.shape[1]), (0, 0)))
        x[nt] = _mm(_NPAD[nt], k)(xp, wp, kw[f"proj_{nt}_b"].reshape(1, HID))

    np_pat = _NPAD["patient"]
    src_f, dstg_f, dsts_f, ep_f = edges["follows"]
    ones_f = jnp.ones((ep_f, HID), jnp.float32)
    deg = _scatter(np_pat, HID, ep_f, _R128["patient"])(
        ones_f, dsts_f, z128["patient"])
    dinv = _ew((HID,), (), L, _f_dinv, np_pat)(deg)

    for l in (1, 2, 3):
        xwn = _mm_scale(np_pat, HID)(x["patient"], kw[f"l{l}_follows_W"],
                                     dinv)
        xwr = _gather(np_pat, HID, ep_f)(xwn, src_f)
        gsc = _scatter(np_pat, HID, ep_f, _R128["patient"])(
            xwr, dsts_f, z128["patient"])

        num = {}
        den = {}
        for rel, sn, dn in _GAT_RELS:
            src, dstg, dsts, ep = edges[rel]
            hs = _mm(_NPAD[sn], HID)(x[sn], kw[f"l{l}_{rel}_Ws"], zb)
            hd = _mm(_NPAD[dn], HID)(x[dn], kw[f"l{l}_{rel}_Wd"], zb)
            hsr = _gather(_NPAD[sn], HID, ep)(hs, src)
            hdr = _gather(_NPAD[dn], HID, ep)(hd, dstg)
            ex = _ew((HID, HID), (HID, HID), L, _f_ex16, ep)(
                hsr, hdr,
                kw[f"l{l}_{rel}_as"].reshape(1, HID),
                kw[f"l{l}_{rel}_ad"].reshape(1, HID))
            nrows = _ew((HID, HID), (), HID, _f_num, ep)(hsr, ex)
            den[rel] = _scatter_c16(_NPAD[dn], ep, _R128[dn])(
                ex, dsts, z128[dn])
            num[rel] = _scatter(_NPAD[dn], HID, ep, _R128[dn])(
                nrows, dsts, z128[dn])

        x = {
            "patient": _ew((HID, HID, L, HID, HID, HID, HID),
                           (HID, HID, HID), HID, _f_cpat, np_pat)(
                gsc, xwn, dinv,
                num["rev_has"], den["rev_has"],
                num["rev_diagnosed"], den["rev_diagnosed"],
                kw[f"l{l}_follows_b"].reshape(1, HID),
                kw[f"l{l}_rev_has_b"].reshape(1, HID),
                kw[f"l{l}_rev_diagnosed_b"].reshape(1, HID)),
            "signature": _ew((HID, HID), (HID,), HID, _f_crelu,
                             _NPAD["signature"])(
                num["has"], den["has"],
                kw[f"l{l}_has_b"].reshape(1, HID)),
            "condition": _ew((HID, HID), (HID,), HID, _f_crelu,
                             _NPAD["condition"])(
                num["diagnosed"], den["diagnosed"],
                kw[f"l{l}_diagnosed_b"].reshape(1, HID)),
        }

    res = []
    for nt in node_types:
        o = _mm(_NPAD[nt], HID)(x[nt], kw[f"out_{nt}_W"],
                                kw[f"out_{nt}_b"].reshape(1, HID))
        res.append(o[:_NREAL[nt]])
    return tuple(res)


# speedup vs baseline: 1.1038x; 1.1038x over previous
import functools

import jax
import jax.numpy as jnp
from jax import lax
from jax.experimental import pallas as pl
from jax.experimental.pallas import tpu as pltpu
from jax.experimental.pallas import tpu_sc as plsc

NC, NS, L = 2, 16, 16
NW = NC * NS
CB = 128
TRASH = 128
HID = 128
NHEAD, CDIM = 4, 32
BN = 512

_mesh = functools.partial(
    plsc.VectorSubcoreMesh, core_axis_name="c", subcore_axis_name="s")


def _ceil_to(x, m):
    return ((x + m - 1) // m) * m


@functools.lru_cache(maxsize=None)
def _gather(N, D, E):
    cnt = E // NW
    n_chunks = cnt // CB

    def body(table, idx, out, idx_v, rows_v, sem):
        wid = lax.axis_index("s") * NC + lax.axis_index("c")
        base = wid * cnt

        def chunk(i, carry):
            off = pl.multiple_of(base + i * CB, CB)
            pltpu.sync_copy(idx.at[pl.ds(off, CB)], idx_v)
            pltpu.async_copy(table.at[idx_v], rows_v, sem).wait()
            pltpu.sync_copy(rows_v, out.at[pl.ds(off, CB)])
            return carry

        lax.fori_loop(0, n_chunks, chunk, 0)

    return pl.kernel(
        body,
        out_type=jax.ShapeDtypeStruct((E, D), jnp.float32),
        mesh=_mesh(),
        scratch_types=[
            pltpu.VMEM((CB,), jnp.int32),
            pltpu.VMEM((CB, D), jnp.float32),
            pltpu.SemaphoreType.DMA,
        ],
    )


@functools.lru_cache(maxsize=None)
def _scatter(N, D, E, R):
    n_ranges = N // R
    assert n_ranges % NC == 0, (N, R)
    n_ri = n_ranges // NC
    cnt = E // NS
    n_chunks = cnt // CB
    out_rpt = R // NS
    zslice = (R + TRASH) // NS

    def body(rows, dst, zeros, out, idx_v, lidx_v, rows_v, acc):
        c = lax.axis_index("c")
        s = lax.axis_index("s")
        for ri in range(n_ri):
            r = ri * NC + c
            base = r * R

            zo = pl.multiple_of(s * zslice, 8)
            pltpu.sync_copy(zeros.at[pl.ds(zo, zslice)],
                            acc.at[pl.ds(zo, zslice)])

            plsc.subcore_barrier()

            def chunk(i, carry):
                off = pl.multiple_of(s * cnt + i * CB, CB)
                pltpu.sync_copy(dst.at[pl.ds(off, CB)], idx_v)
                pltpu.sync_copy(rows.at[pl.ds(off, CB)], rows_v)

                def fix(j, carry2):
                    dv = idx_v[pl.ds(j * L, L)]
                    lv = dv - base
                    inb = (lv >= 0) & (lv < R)
                    lidx_v[pl.ds(j * L, L)] = jnp.where(inb, lv, R + s)
                    return carry2

                lax.fori_loop(0, CB // L, fix, 0)
                pltpu.sync_copy(rows_v, acc.at[lidx_v], add=True)
                return carry

            lax.fori_loop(0, n_chunks, chunk, 0)

            plsc.subcore_barrier()

            o = pl.multiple_of(s * out_rpt, 8)
            oo = pl.multiple_of(base + s * out_rpt, 8)
            pltpu.sync_copy(acc.at[pl.ds(o, out_rpt)],
                            out.at[pl.ds(oo, out_rpt)])

            plsc.subcore_barrier()

    return pl.kernel(
        body,
        out_type=jax.ShapeDtypeStruct((N, D), jnp.float32),
        mesh=_mesh(),
        scratch_types=[
            pltpu.VMEM((CB,), jnp.int32),
            pltpu.VMEM((CB,), jnp.int32),
            pltpu.VMEM((CB, D), jnp.float32),
            pltpu.VMEM_SHARED((R + TRASH, D), jnp.float32),
        ],
    )


@functools.lru_cache(maxsize=None)
def _scatter_ones(N, E, R):
    n_ranges = N // R
    assert n_ranges % NC == 0, (N, R)
    n_ri = n_ranges // NC
    cnt = E // NS
    n_chunks = cnt // CB
    out_rpt = R // NS
    zslice = (R + TRASH) // NS

    def body(ones, dst, zeros, out, idx_v, lidx_v, rows_v, acc):
        c = lax.axis_index("c")
        s = lax.axis_index("s")
        pltpu.sync_copy(ones, rows_v)
        for ri in range(n_ri):
            r = ri * NC + c
            base = r * R

            zo = pl.multiple_of(s * zslice, 8)
            pltpu.sync_copy(zeros.at[pl.ds(zo, zslice)],
                            acc.at[pl.ds(zo, zslice)])

            plsc.subcore_barrier()

            def chunk(i, carry):
                off = pl.multiple_of(s * cnt + i * CB, CB)
                pltpu.sync_copy(dst.at[pl.ds(off, CB)], idx_v)

                def fix(j, carry2):
                    dv = idx_v[pl.ds(j * L, L)]
                    lv = dv - base
                    inb = (lv >= 0) & (lv < R)
                    lidx_v[pl.ds(j * L, L)] = jnp.where(inb, lv, R + s)
                    return carry2

                lax.fori_loop(0, CB // L, fix, 0)
                pltpu.sync_copy(rows_v, acc.at[lidx_v], add=True)
                return carry

            lax.fori_loop(0, n_chunks, chunk, 0)

            plsc.subcore_barrier()

            o = pl.multiple_of(s * out_rpt, 8)
            oo = pl.multiple_of(base + s * out_rpt, 8)
            pltpu.sync_copy(acc.at[pl.ds(o, out_rpt)],
                            out.at[pl.ds(oo, out_rpt)])

            plsc.subcore_barrier()

    return pl.kernel(
        body,
        out_type=jax.ShapeDtypeStruct((N, HID), jnp.float32),
        mesh=_mesh(),
        scratch_types=[
            pltpu.VMEM((CB,), jnp.int32),
            pltpu.VMEM((CB,), jnp.int32),
            pltpu.VMEM((CB, HID), jnp.float32),
            pltpu.VMEM_SHARED((R + TRASH, HID), jnp.float32),
        ],
    )


def _mm_block(x_ref, w_ref, b_ref, o_ref):
    o_ref[...] = jnp.dot(x_ref[...], w_ref[...],
                         preferred_element_type=jnp.float32) + b_ref[...]


@functools.lru_cache(maxsize=None)
def _mm(N, K):
    return pl.pallas_call(
        _mm_block,
        grid=(N // BN,),
        in_specs=[pl.BlockSpec((BN, K), lambda i: (i, 0)),
                  pl.BlockSpec((K, HID), lambda i: (0, 0)),
                  pl.BlockSpec((1, HID), lambda i: (0, 0))],
        out_specs=pl.BlockSpec((BN, HID), lambda i: (i, 0)),
        out_shape=jax.ShapeDtypeStruct((N, HID), jnp.float32),
    )


def _mms_block(x_ref, w_ref, s_ref, o_ref):
    o_ref[...] = jnp.dot(x_ref[...], w_ref[...],
                         preferred_element_type=jnp.float32) * s_ref[:, 0:1]


@functools.lru_cache(maxsize=None)
def _mm_scale(N, K):
    return pl.pallas_call(
        _mms_block,
        grid=(N // BN,),
        in_specs=[pl.BlockSpec((BN, K), lambda i: (i, 0)),
                  pl.BlockSpec((K, HID), lambda i: (0, 0)),
                  pl.BlockSpec((BN, L), lambda i: (i, 0))],
        out_specs=pl.BlockSpec((BN, HID), lambda i: (i, 0)),
        out_shape=jax.ShapeDtypeStruct((N, HID), jnp.float32),
    )


def _ew(d_in, d_par, d_out, fn, N):
    def block(*refs):
        o_ref = refs[-1]
        o_ref[...] = fn(*[r[...] for r in refs[:-1]])

    return pl.pallas_call(
        block,
        grid=(N // BN,),
        in_specs=([pl.BlockSpec((BN, d), lambda i: (i, 0)) for d in d_in]
                  + [pl.BlockSpec((1, d), lambda i: (0, 0)) for d in d_par]),
        out_specs=pl.BlockSpec((BN, d_out), lambda i: (i, 0)),
        out_shape=jax.ShapeDtypeStruct((N, d_out), jnp.float32),
    )


def _heads_reduce(rows, a):
    p = rows * a
    return [jnp.sum(p[:, i * CDIM:(i + 1) * CDIM], axis=1, keepdims=True)
            for i in range(NHEAD)]


def _heads_expand(v):
    return jnp.concatenate(
        [jnp.broadcast_to(v[:, i:i + 1], (v.shape[0], CDIM))
         for i in range(NHEAD)], axis=1)


def _f_ex16(hsr, hdr, a_s, a_d):
    es = _heads_reduce(hsr, a_s)
    ed = _heads_reduce(hdr, a_d)
    cols = [s + d for s, d in zip(es, ed)]
    cols.append(jnp.zeros((hsr.shape[0], HID - NHEAD), jnp.float32))
    a = jnp.concatenate(cols, axis=1)
    return jnp.exp(jnp.where(a > 0, a, 0.2 * a))


def _f_num(hsr, ex):
    return hsr * _heads_expand(ex)


def _f_dinv(deg):
    return lax.rsqrt(deg[:, :L] + 1.0)


def _gat_norm(num, den):
    return num * _heads_expand(1.0 / (den + 1e-16))


def _f_cpat(s_gcn, xwn, di, num_rh, den_rh, num_rd, den_rd, b1, b2, b3):
    d = di[:, 0:1]
    return jnp.maximum(
        (s_gcn + xwn) * d + _gat_norm(num_rh, den_rh)
        + _gat_norm(num_rd, den_rd) + b1 + b2 + b3, 0.0)


def _f_crelu(num, den, b):
    return jnp.maximum(_gat_norm(num, den) + b, 0.0)


_NPAD = {"patient": 51200, "signature": 20480, "condition": 1024}
_NREAL = {"patient": 50000, "signature": 20000, "condition": 500}
_R128 = {"patient": 12800, "signature": 10240, "condition": 512}
_GAT_RELS = [("has", "patient", "signature"),
             ("rev_has", "signature", "patient"),
             ("diagnosed", "patient", "condition"),
             ("rev_diagnosed", "condition", "patient")]


def kernel(x_patient, x_signature, x_condition, edge_follows, edge_has, edge_rev_has, edge_diagnosed, edge_rev_diagnosed, proj_patient_W, proj_patient_b, out_patient_W, out_patient_b, proj_signature_W, proj_signature_b, out_signature_W, out_signature_b, proj_condition_W, proj_condition_b, out_condition_W, out_condition_b, l1_follows_W, l1_follows_b, l1_has_Ws, l1_has_Wd, l1_has_as, l1_has_ad, l1_has_b, l1_rev_has_Ws, l1_rev_has_Wd, l1_rev_has_as, l1_rev_has_ad, l1_rev_has_b, l1_diagnosed_Ws, l1_diagnosed_Wd, l1_diagnosed_as, l1_diagnosed_ad, l1_diagnosed_b, l1_rev_diagnosed_Ws, l1_rev_diagnosed_Wd, l1_rev_diagnosed_as, l1_rev_diagnosed_ad, l1_rev_diagnosed_b, l2_follows_W, l2_follows_b, l2_has_Ws, l2_has_Wd, l2_has_as, l2_has_ad, l2_has_b, l2_rev_has_Ws, l2_rev_has_Wd, l2_rev_has_as, l2_rev_has_ad, l2_rev_has_b, l2_diagnosed_Ws, l2_diagnosed_Wd, l2_diagnosed_as, l2_diagnosed_ad, l2_diagnosed_b, l2_rev_diagnosed_Ws, l2_rev_diagnosed_Wd, l2_rev_diagnosed_as, l2_rev_diagnosed_ad, l2_rev_diagnosed_b, l3_follows_W, l3_follows_b, l3_has_Ws, l3_has_Wd, l3_has_as, l3_has_ad, l3_has_b, l3_rev_has_Ws, l3_rev_has_Wd, l3_rev_has_as, l3_rev_has_ad, l3_rev_has_b, l3_diagnosed_Ws, l3_diagnosed_Wd, l3_diagnosed_as, l3_diagnosed_ad, l3_diagnosed_b, l3_rev_diagnosed_Ws, l3_rev_diagnosed_Wd, l3_rev_diagnosed_as, l3_rev_diagnosed_ad, l3_rev_diagnosed_b):
    kw = dict(locals())
    node_types = ["patient", "signature", "condition"]

    edges = {}
    for rel in ("follows", "has", "rev_has", "diagnosed", "rev_diagnosed"):
        ei = kw[f"edge_{rel}"].astype(jnp.int32)
        e = ei.shape[1]
        ep = _ceil_to(e, NW * CB)
        src = jnp.pad(ei[0], (0, ep - e))
        dstg = jnp.pad(ei[1], (0, ep - e))
        dsts = jnp.pad(ei[1], (0, ep - e), constant_values=-1)
        edges[rel] = (src, dstg, dsts, ep)

    zb = jnp.zeros((1, HID), jnp.float32)
    z128 = {nt: jnp.zeros((_R128[nt] + TRASH, HID), jnp.float32)
            for nt in node_types}

    x = {}
    for nt in node_types:
        xin = kw[f"x_{nt}"]
        k = _ceil_to(xin.shape[1], HID)
        xp = jnp.pad(xin, ((0, _NPAD[nt] - xin.shape[0]),
                           (0, k - xin.shape[1])))
        wp = jnp.pad(kw[f"proj_{nt}_W"], ((0, k - xin.shape[1]), (0, 0)))
        x[nt] = _mm(_NPAD[nt], k)(xp, wp, kw[f"proj_{nt}_b"].reshape(1, HID))

    np_pat = _NPAD["patient"]
    src_f, dstg_f, dsts_f, ep_f = edges["follows"]
    ones_blk = jnp.ones((CB, HID), jnp.float32)
    deg = _scatter_ones(np_pat, ep_f, _R128["patient"])(
        ones_blk, dsts_f, z128["patient"])
    dinv = _ew((HID,), (), L, _f_dinv, np_pat)(deg)

    for l in (1, 2, 3):
        xwn = _mm_scale(np_pat, HID)(x["patient"], kw[f"l{l}_follows_W"],
                                     dinv)
        xwr = _gather(np_pat, HID, ep_f)(xwn, src_f)
        gsc = _scatter(np_pat, HID, ep_f, _R128["patient"])(
            xwr, dsts_f, z128["patient"])

        num = {}
        den = {}
        for rel, sn, dn in _GAT_RELS:
            src, dstg, dsts, ep = edges[rel]
            hs = _mm(_NPAD[sn], HID)(x[sn], kw[f"l{l}_{rel}_Ws"], zb)
            hd = _mm(_NPAD[dn], HID)(x[dn], kw[f"l{l}_{rel}_Wd"], zb)
            hsr = _gather(_NPAD[sn], HID, ep)(hs, src)
            hdr = _gather(_NPAD[dn], HID, ep)(hd, dstg)
            ex = _ew((HID, HID), (HID, HID), HID, _f_ex16, ep)(
                hsr, hdr,
                kw[f"l{l}_{rel}_as"].reshape(1, HID),
                kw[f"l{l}_{rel}_ad"].reshape(1, HID))
            nrows = _ew((HID, HID), (), HID, _f_num, ep)(hsr, ex)
            den[rel] = _scatter(_NPAD[dn], HID, ep, _R128[dn])(
                ex, dsts, z128[dn])
            num[rel] = _scatter(_NPAD[dn], HID, ep, _R128[dn])(
                nrows, dsts, z128[dn])

        x = {
            "patient": _ew((HID, HID, L, HID, HID, HID, HID),
                           (HID, HID, HID), HID, _f_cpat, np_pat)(
                gsc, xwn, dinv,
                num["rev_has"], den["rev_has"],
                num["rev_diagnosed"], den["rev_diagnosed"],
                kw[f"l{l}_follows_b"].reshape(1, HID),
                kw[f"l{l}_rev_has_b"].reshape(1, HID),
                kw[f"l{l}_rev_diagnosed_b"].reshape(1, HID)),
            "signature": _ew((HID, HID), (HID,), HID, _f_crelu,
                             _NPAD["signature"])(
                num["has"], den["has"],
                kw[f"l{l}_has_b"].reshape(1, HID)),
            "condition": _ew((HID, HID), (HID,), HID, _f_crelu,
                             _NPAD["condition"])(
                num["diagnosed"], den["diagnosed"],
                kw[f"l{l}_diagnosed_b"].reshape(1, HID)),
        }

    res = []
    for nt in node_types:
        o = _mm(_NPAD[nt], HID)(x[nt], kw[f"out_{nt}_W"],
                                kw[f"out_{nt}_b"].reshape(1, HID))
        res.append(o[:_NREAL[nt]])
    return tuple(res)
